# Initial kernel scaffold; baseline (speedup 1.0000x reference)
import jax
import jax.numpy as jnp
from jax.experimental import pallas as pl


def kernel(list_neigh, Imagetype_map, atom_type, ImageDR, nghost, c_param_2, c_param_3, fit_w0, fit_b0, fit_w1, fit_b1):
    raise NotImplementedError("write your pallas kernel here")


# speedup vs baseline: 2.4244x; 2.4244x over previous
import functools

import jax
import jax.numpy as jnp
from jax.experimental import pallas as pl

_B = 1
_N = 10000
_T = 2
_M = 128
_K2 = 9
_K3 = 9
_P = 5
_H = 100
_RC_R = 8.0
_RC_A = 4.0
_BN = 200


def _nep_block(dr_ref, ln_ref, it_ref, c2m_ref, c3m_ref, w0_ref, b0_ref,
               w1_ref, b1_ref, ei_ref, etot_ref):
    f32 = jnp.float32
    r = dr_ref[0]
    x = dr_ref[1]
    y = dr_ref[2]
    z = dr_ref[3]
    ln = ln_ref[...]

    mask_f = ((ln > 0) & (r > 1e-5)).astype(f32)
    c8 = jnp.cos(r * (jnp.pi / _RC_R))
    fc2 = (0.5 * (c8 + 1.0)) * (r < _RC_R).astype(f32)
    c4 = 2.0 * c8 * c8 - 1.0
    fc3 = (0.5 * (c4 + 1.0)) * (r < _RC_A).astype(f32)
    mask_a = mask_f * (r < _RC_A).astype(f32)
    fm2 = fc2 * mask_f
    fm3 = fc3 * mask_a

    x2 = 2.0 * (r * (1.0 / _RC_R) - 1.0) ** 2 - 1.0
    x3 = 2.0 * (r * (1.0 / _RC_A) - 1.0) ** 2 - 1.0

    rinv = 1.0 / jnp.maximum(r, 1e-6)
    ux = x * rinv
    uy = y * rinv
    uz = z * rinv
    phi0 = 0.5 * (3.0 * uz * uz - (ux * ux + uy * uy + uz * uz))
    geom = (ux, uy, uz, phi0, ux * uz, uy * uz, ux * ux - uy * uy, ux * uy)

    half0 = (jax.lax.broadcasted_iota(jnp.int32, r.shape, 1) < (_M // 2)).astype(f32)

    def cheb_columns(xc, fm, weights):
        cols = [[] for _ in weights]
        t_prev = None
        t_cur = None
        for k in range(_K2):
            if k == 0:
                tk = jnp.ones_like(xc)
            elif k == 1:
                tk = xc
            else:
                tk = 2.0 * xc * t_cur - t_prev
            t_prev, t_cur = (t_cur, tk) if k >= 1 else (tk, tk)
            fk = (0.5 * tk + 0.5) * fm
            for gi, g in enumerate(weights):
                a = fk if g is None else fk * g
                s_all = jnp.sum(a, axis=1, keepdims=True)
                s0 = jnp.sum(a * half0, axis=1, keepdims=True)
                cols[gi].append((s0, s_all - s0))
        return cols

    (rad_cols,) = cheb_columns(x2, fm2, (None,))
    ang_cols = cheb_columns(x3, fm3, geom)

    def assemble(cols):
        return jnp.concatenate([c[0] for c in cols] + [c[1] for c in cols], axis=1)

    it = it_ref[...]
    sel = it > 0.5

    def typed_mm(mat, w_ref):
        m0 = jax.lax.dot_general(mat, w_ref[0], (((1,), (0,)), ((), ())),
                                 preferred_element_type=f32)
        m1 = jax.lax.dot_general(mat, w_ref[1], (((1,), (0,)), ((), ())),
                                 preferred_element_type=f32)
        return jnp.where(sel, m1, m0)

    q2 = typed_mm(assemble(rad_cols), c2m_ref)

    s_list = [typed_mm(assemble(c), c3m_ref) for c in ang_cols]
    q31 = s_list[0] ** 2 + s_list[1] ** 2 + s_list[2] ** 2
    q32 = (s_list[3] ** 2 + 3.0 * (s_list[4] ** 2) + 3.0 * (s_list[5] ** 2)
           + 0.75 * (s_list[6] ** 2) + 3.0 * (s_list[7] ** 2))

    q = jnp.concatenate([q2, q31, q32], axis=1)

    z0 = jax.lax.dot_general(q, w0_ref[0], (((1,), (0,)), ((), ())),
                             preferred_element_type=f32) + b0_ref[0]
    z1 = jax.lax.dot_general(q, w0_ref[1], (((1,), (0,)), ((), ())),
                             preferred_element_type=f32) + b0_ref[1]
    h = jnp.tanh(jnp.where(sel, z1, z0))
    e0 = jax.lax.dot_general(h, w1_ref[0], (((1,), (0,)), ((), ())),
                             preferred_element_type=f32) + b1_ref[0, 0]
    e1 = jax.lax.dot_general(h, w1_ref[1], (((1,), (0,)), ((), ())),
                             preferred_element_type=f32) + b1_ref[1, 0]
    ei = jnp.where(sel, e1, e0)
    ei_ref[...] = ei

    @pl.when(pl.program_id(0) == 0)
    def _init():
        etot_ref[...] = jnp.zeros_like(etot_ref)

    etot_ref[...] += jnp.sum(ei).reshape(1, 1)


@functools.partial(jax.jit, static_argnames=())
def _run(dr_t, ln, it_f, c2m, c3m, w0, b0, w1, b1):
    nb = _N // _BN
    grid = (nb,)
    kern = pl.pallas_call(
        _nep_block,
        grid=grid,
        in_specs=[
            pl.BlockSpec((4, _BN, _M), lambda i: (0, i, 0)),
            pl.BlockSpec((_BN, _M), lambda i: (i, 0)),
            pl.BlockSpec((_BN, 1), lambda i: (i, 0)),
            pl.BlockSpec((_T, 2 * _K2, _P), lambda i: (0, 0, 0)),
            pl.BlockSpec((_T, 2 * _K3, _P), lambda i: (0, 0, 0)),
            pl.BlockSpec((_T, _P * 3, _H), lambda i: (0, 0, 0)),
            pl.BlockSpec((_T, _H), lambda i: (0, 0)),
            pl.BlockSpec((_T, _H, 1), lambda i: (0, 0, 0)),
            pl.BlockSpec((_T, 1), lambda i: (0, 0)),
        ],
        out_specs=[
            pl.BlockSpec((_BN, 1), lambda i: (i, 0)),
            pl.BlockSpec((1, 1), lambda i: (0, 0)),
        ],
        out_shape=[
            jax.ShapeDtypeStruct((_N, 1), jnp.float32),
            jax.ShapeDtypeStruct((1, 1), jnp.float32),
        ],
    )
    ei, etot = kern(dr_t, ln, it_f, c2m, c3m, w0, b0, w1, b1)
    return ei, etot


def kernel(list_neigh, Imagetype_map, atom_type, ImageDR, nghost, c_param_2,
           c_param_3, fit_w0, fit_b0, fit_w1, fit_b1):
    dr_t = jnp.transpose(ImageDR[0], (2, 0, 1))
    ln = list_neigh[0]
    it_f = Imagetype_map.astype(jnp.float32).reshape(_N, 1)
    c2m = jnp.transpose(c_param_2, (0, 1, 3, 2)).reshape(_T, 2 * _K2, _P)
    c3m = jnp.transpose(c_param_3, (0, 1, 3, 2)).reshape(_T, 2 * _K3, _P)
    ei, etot = _run(dr_t, ln, it_f, c2m, c3m, fit_w0, fit_b0, fit_w1, fit_b1)
    Ei = ei.reshape(_B, _N, 1)
    Etot = etot.reshape(_B)
    return Etot, Ei


# speedup vs baseline: 4.4247x; 1.8250x over previous
import functools

import jax
import jax.numpy as jnp
from jax.experimental import pallas as pl

_B = 1
_N = 10000
_T = 2
_M = 128
_NM = 64
_K = 9
_P = 5
_H = 100
_RC_R = 8.0
_RC_A = 4.0
_BN = 256
_NP = ((_N + _BN - 1) // _BN) * _BN


def _nep_block(drt_ref, lnt_ref, it_ref, c2_ref, c3_ref, w0t_ref, b0t_ref,
               w1t_ref, b1_ref, ei_ref, etot_ref):
    f32 = jnp.float32
    it_row = it_ref[...] > 0.5

    def coeff(cref, h, p, k):
        return jnp.where(it_row, cref[1, h, p, k], cref[0, h, p, k])

    def cheb_contract(xc, fm, cref, h):
        acc = [None] * _P
        t_prev = None
        t_cur = None
        for k in range(_K):
            if k == 0:
                tk = jnp.ones_like(xc)
            elif k == 1:
                tk = xc
            else:
                tk = 2.0 * xc * t_cur - t_prev
            t_prev, t_cur = (t_cur, tk) if k >= 1 else (tk, tk)
            fk = (0.5 * tk + 0.5) * fm
            for p in range(_P):
                term = fk * coeff(cref, h, p, k)
                acc[p] = term if acc[p] is None else acc[p] + term
        return acc

    def rsum(a):
        return jnp.sum(a, axis=0, keepdims=True)

    def half(h):
        sl = slice(h * _NM, (h + 1) * _NM)
        r = drt_ref[0, sl, :]
        x = drt_ref[1, sl, :]
        y = drt_ref[2, sl, :]
        z = drt_ref[3, sl, :]
        ln = lnt_ref[sl, :]
        mask_f = ((ln > 0) & (r > 1e-5)).astype(f32)
        c8 = jnp.cos(r * (jnp.pi / _RC_R))
        in_r = (r < _RC_R).astype(f32)
        in_a = (r < _RC_A).astype(f32)
        fm2 = (0.5 * (c8 + 1.0)) * in_r * mask_f
        c4 = 2.0 * c8 * c8 - 1.0
        fm3 = (0.5 * (c4 + 1.0)) * in_a * mask_f
        x2 = 2.0 * (r * (1.0 / _RC_R) - 1.0) ** 2 - 1.0
        x3 = 2.0 * (r * (1.0 / _RC_A) - 1.0) ** 2 - 1.0
        rinv = 1.0 / jnp.maximum(r, 1e-6)
        ux = x * rinv
        uy = y * rinv
        uz = z * rinv
        phi0 = 0.5 * (3.0 * uz * uz - (ux * ux + uy * uy + uz * uz))
        geom = (ux, uy, uz, phi0, ux * uz, uy * uz, ux * ux - uy * uy, ux * uy)
        g2 = cheb_contract(x2, fm2, c2_ref, h)
        rad = [rsum(g2[p]) for p in range(_P)]
        g3 = cheb_contract(x3, fm3, c3_ref, h)
        ang = [[rsum(g3[p] * g) for g in geom] for p in range(_P)]
        return rad, ang

    rad0, ang0 = half(0)
    rad1, ang1 = half(1)

    q_rows = [rad0[p] + rad1[p] for p in range(_P)]
    s = [[ang0[p][g] + ang1[p][g] for g in range(8)] for p in range(_P)]
    q_rows += [s[p][0] ** 2 + s[p][1] ** 2 + s[p][2] ** 2 for p in range(_P)]
    q_rows += [s[p][3] ** 2 + 3.0 * s[p][4] ** 2 + 3.0 * s[p][5] ** 2
               + 0.75 * s[p][6] ** 2 + 3.0 * s[p][7] ** 2 for p in range(_P)]
    q = jnp.concatenate(q_rows, axis=0)

    f32 = jnp.float32
    z0 = jax.lax.dot_general(w0t_ref[0], q, (((1,), (0,)), ((), ())),
                             preferred_element_type=f32) + b0t_ref[0]
    z1 = jax.lax.dot_general(w0t_ref[1], q, (((1,), (0,)), ((), ())),
                             preferred_element_type=f32) + b0t_ref[1]
    hact = jnp.tanh(jnp.where(it_row, z1, z0))
    e0 = jax.lax.dot_general(w1t_ref[0], hact, (((1,), (0,)), ((), ())),
                             preferred_element_type=f32) + b1_ref[0, 0]
    e1 = jax.lax.dot_general(w1t_ref[1], hact, (((1,), (0,)), ((), ())),
                             preferred_element_type=f32) + b1_ref[1, 0]
    ei = jnp.where(it_row, e1, e0)
    ei_ref[...] = ei

    @pl.when(pl.program_id(0) == 0)
    def _init():
        etot_ref[...] = jnp.zeros_like(etot_ref)

    idx = jax.lax.broadcasted_iota(jnp.int32, (1, _BN), 1) + pl.program_id(0) * _BN
    etot_ref[...] += jnp.sum(jnp.where(idx < _N, ei, 0.0)).reshape(1, 1)


@jax.jit
def _run(drt, lnt, itr, c2, c3, w0t, b0t, w1t, b1):
    nb = _NP // _BN
    kern = pl.pallas_call(
        _nep_block,
        grid=(nb,),
        in_specs=[
            pl.BlockSpec((4, _M, _BN), lambda i: (0, 0, i)),
            pl.BlockSpec((_M, _BN), lambda i: (0, i)),
            pl.BlockSpec((1, _BN), lambda i: (0, i)),
            pl.BlockSpec((_T, _T, _P, _K), lambda i: (0, 0, 0, 0)),
            pl.BlockSpec((_T, _T, _P, _K), lambda i: (0, 0, 0, 0)),
            pl.BlockSpec((_T, _H, _P * 3), lambda i: (0, 0, 0)),
            pl.BlockSpec((_T, _H, 1), lambda i: (0, 0, 0)),
            pl.BlockSpec((_T, 1, _H), lambda i: (0, 0, 0)),
            pl.BlockSpec((_T, 1), lambda i: (0, 0)),
        ],
        out_specs=[
            pl.BlockSpec((1, _BN), lambda i: (0, i)),
            pl.BlockSpec((1, 1), lambda i: (0, 0)),
        ],
        out_shape=[
            jax.ShapeDtypeStruct((1, _NP), jnp.float32),
            jax.ShapeDtypeStruct((1, 1), jnp.float32),
        ],
    )
    return kern(drt, lnt, itr, c2, c3, w0t, b0t, w1t, b1)


def kernel(list_neigh, Imagetype_map, atom_type, ImageDR, nghost, c_param_2,
           c_param_3, fit_w0, fit_b0, fit_w1, fit_b1):
    pad_n = _NP - _N
    drt = jnp.pad(jnp.transpose(ImageDR[0], (2, 1, 0)), ((0, 0), (0, 0), (0, pad_n)))
    lnt = jnp.pad(jnp.transpose(list_neigh[0], (1, 0)), ((0, 0), (0, pad_n)))
    itr = jnp.pad(Imagetype_map.astype(jnp.float32), (0, pad_n)).reshape(1, _NP)
    w0t = jnp.transpose(fit_w0, (0, 2, 1))
    b0t = fit_b0[..., None]
    w1t = jnp.transpose(fit_w1, (0, 2, 1))
    ei, etot = _run(drt, lnt, itr, c_param_2, c_param_3, w0t, b0t, w1t, fit_b1)
    Ei = ei[0, :_N].reshape(_B, _N, 1)
    Etot = etot.reshape(_B)
    return Etot, Ei


# speedup vs baseline: 5.2219x; 1.1802x over previous
import functools

import jax
import jax.numpy as jnp
from jax.experimental import pallas as pl

_B = 1
_N = 10000
_T = 2
_M = 128
_NM = 64
_K = 9
_P = 5
_H = 100
_RC_R = 8.0
_RC_A = 4.0
_BN = 512
_NP = ((_N + _BN - 1) // _BN) * _BN

_PI2 = float(jnp.pi) ** 2
_COS_SQRT_COEFS = []
_fact = 1.0
for _k in range(11):
    if _k > 0:
        _fact *= (2 * _k - 1) * (2 * _k)
    _COS_SQRT_COEFS.append((-_PI2) ** _k / _fact)


def _cos_pi_sqrt(w):
    acc = jnp.full_like(w, _COS_SQRT_COEFS[-1])
    for c in _COS_SQRT_COEFS[-2::-1]:
        acc = acc * w + c
    return acc


def _nep_block(drt_ref, lnt_ref, it_ref, c2_ref, c3_ref, w0t_ref, b0t_ref,
               w1t_ref, b1_ref, ei_ref, etot_ref):
    f32 = jnp.float32
    it_row = it_ref[...] > 0.5

    def coeff(cref, h, p, k):
        return jnp.where(it_row, cref[1, h, p, k], cref[0, h, p, k])

    def cheb_contract(xc, cref, h):
        acc = [None] * _P
        t_prev = None
        t_cur = None
        for k in range(_K):
            if k == 0:
                tk = jnp.ones_like(xc)
            elif k == 1:
                tk = xc
            else:
                tk = 2.0 * xc * t_cur - t_prev
            t_prev, t_cur = (t_cur, tk) if k >= 1 else (tk, tk)
            for p in range(_P):
                term = tk * coeff(cref, h, p, k)
                acc[p] = term if acc[p] is None else acc[p] + term
        return acc

    def rsum(a):
        return jnp.sum(a, axis=0, keepdims=True)

    def half(h):
        sl = slice(h * _NM, (h + 1) * _NM)
        r = drt_ref[0, sl, :]
        x = drt_ref[1, sl, :]
        y = drt_ref[2, sl, :]
        z = drt_ref[3, sl, :]
        ln = lnt_ref[sl, :]
        mask_f = ((ln > 0) & (r > 1e-5)).astype(f32)
        w2 = (r * (1.0 / _RC_R) - 1.0) ** 2
        w3 = (r * (1.0 / _RC_A) - 1.0) ** 2
        c8 = -_cos_pi_sqrt(w2)
        in_r = (r < _RC_R).astype(f32)
        in_a = (r < _RC_A).astype(f32)
        fm2 = (0.5 * (c8 + 1.0)) * in_r * mask_f
        c4 = 2.0 * c8 * c8 - 1.0
        fm3 = (0.5 * (c4 + 1.0)) * in_a * mask_f
        x2 = 2.0 * w2 - 1.0
        x3 = 2.0 * w3 - 1.0
        rinv = 1.0 / jnp.maximum(r, 1e-6)
        ux = x * rinv
        uy = y * rinv
        uz = z * rinv
        phi0 = 0.5 * (3.0 * uz * uz - (ux * ux + uy * uy + uz * uz))
        geom = (ux, uy, uz, phi0, ux * uz, uy * uz, ux * ux - uy * uy, ux * uy)
        g2 = cheb_contract(x2, c2_ref, h)
        rad = [rsum(fm2 * g2[p]) for p in range(_P)]
        g3 = cheb_contract(x3, c3_ref, h)
        fg = [fm3 * g for g in geom]
        ang = [[rsum(g3[p] * f) for f in fg] for p in range(_P)]
        return rad, ang

    rad0, ang0 = half(0)
    rad1, ang1 = half(1)

    q_rows = [rad0[p] + rad1[p] for p in range(_P)]
    s = [[ang0[p][g] + ang1[p][g] for g in range(8)] for p in range(_P)]
    q_rows += [s[p][0] ** 2 + s[p][1] ** 2 + s[p][2] ** 2 for p in range(_P)]
    q_rows += [s[p][3] ** 2 + 3.0 * s[p][4] ** 2 + 3.0 * s[p][5] ** 2
               + 0.75 * s[p][6] ** 2 + 3.0 * s[p][7] ** 2 for p in range(_P)]
    q = jnp.concatenate(q_rows, axis=0)

    f32 = jnp.float32
    z0 = jax.lax.dot_general(w0t_ref[0], q, (((1,), (0,)), ((), ())),
                             preferred_element_type=f32) + b0t_ref[0]
    z1 = jax.lax.dot_general(w0t_ref[1], q, (((1,), (0,)), ((), ())),
                             preferred_element_type=f32) + b0t_ref[1]
    hact = jnp.tanh(jnp.where(it_row, z1, z0))
    e0 = jax.lax.dot_general(w1t_ref[0], hact, (((1,), (0,)), ((), ())),
                             preferred_element_type=f32) + b1_ref[0, 0]
    e1 = jax.lax.dot_general(w1t_ref[1], hact, (((1,), (0,)), ((), ())),
                             preferred_element_type=f32) + b1_ref[1, 0]
    ei = jnp.where(it_row, e1, e0)
    ei_ref[...] = ei

    @pl.when(pl.program_id(0) == 0)
    def _init():
        etot_ref[...] = jnp.zeros_like(etot_ref)

    idx = jax.lax.broadcasted_iota(jnp.int32, (1, _BN), 1) + pl.program_id(0) * _BN
    etot_ref[...] += jnp.sum(jnp.where(idx < _N, ei, 0.0)).reshape(1, 1)


@jax.jit
def _run(drt, lnt, itr, c2, c3, w0t, b0t, w1t, b1):
    nb = _NP // _BN
    kern = pl.pallas_call(
        _nep_block,
        grid=(nb,),
        in_specs=[
            pl.BlockSpec((4, _M, _BN), lambda i: (0, 0, i)),
            pl.BlockSpec((_M, _BN), lambda i: (0, i)),
            pl.BlockSpec((1, _BN), lambda i: (0, i)),
            pl.BlockSpec((_T, _T, _P, _K), lambda i: (0, 0, 0, 0)),
            pl.BlockSpec((_T, _T, _P, _K), lambda i: (0, 0, 0, 0)),
            pl.BlockSpec((_T, _H, _P * 3), lambda i: (0, 0, 0)),
            pl.BlockSpec((_T, _H, 1), lambda i: (0, 0, 0)),
            pl.BlockSpec((_T, 1, _H), lambda i: (0, 0, 0)),
            pl.BlockSpec((_T, 1), lambda i: (0, 0)),
        ],
        out_specs=[
            pl.BlockSpec((1, _BN), lambda i: (0, i)),
            pl.BlockSpec((1, 1), lambda i: (0, 0)),
        ],
        out_shape=[
            jax.ShapeDtypeStruct((1, _NP), jnp.float32),
            jax.ShapeDtypeStruct((1, 1), jnp.float32),
        ],
    )
    return kern(drt, lnt, itr, c2, c3, w0t, b0t, w1t, b1)


def kernel(list_neigh, Imagetype_map, atom_type, ImageDR, nghost, c_param_2,
           c_param_3, fit_w0, fit_b0, fit_w1, fit_b1):
    pad_n = _NP - _N

    def fold(c):
        cp = 0.5 * c
        return cp.at[..., 0].add(0.5 * jnp.sum(c, axis=-1))

    c_param_2 = fold(c_param_2)
    c_param_3 = fold(c_param_3)
    drt = jnp.pad(jnp.transpose(ImageDR[0], (2, 1, 0)), ((0, 0), (0, 0), (0, pad_n)))
    lnt = jnp.pad(jnp.transpose(list_neigh[0], (1, 0)), ((0, 0), (0, pad_n)))
    itr = jnp.pad(Imagetype_map.astype(jnp.float32), (0, pad_n)).reshape(1, _NP)
    w0t = jnp.transpose(fit_w0, (0, 2, 1))
    b0t = fit_b0[..., None]
    w1t = jnp.transpose(fit_w1, (0, 2, 1))
    ei, etot = _run(drt, lnt, itr, c_param_2, c_param_3, w0t, b0t, w1t, fit_b1)
    Ei = ei[0, :_N].reshape(_B, _N, 1)
    Etot = etot.reshape(_B)
    return Etot, Ei
